# baseline (speedup 1.0000x reference)
import functools

import jax
import jax.numpy as jnp
from jax import lax
from jax.experimental import pallas as pl
from jax.experimental.pallas import tpu as pltpu
from jax.experimental.pallas import tpu_sc as plsc

N = 16384
BASE_DIM = 128
VOCAB = 1000000
EMB = 64
OUT_DIM = 128

_NC = 2
_NS = 16
_NW = _NC * _NS
_L = 16
_WIN = 384
_NBUF = 3
_NWIN = (VOCAB + _WIN - 1) // _WIN
_WPT = (_NWIN + _NW - 1) // _NW
_LAST_FULL = (VOCAB + 127) // 128 * 128 // _WIN
_NGRP = N // _L
_SLOTS = 128
_LAG = 64


@functools.cache
def _make_sc_gather():
    @functools.partial(
        pl.kernel,
        mesh=plsc.VectorSubcoreMesh(core_axis_name="c", subcore_axis_name="s"),
        out_type=jax.ShapeDtypeStruct((N, EMB), jnp.float32),
        compiler_params=pltpu.CompilerParams(needs_layout_passes=False),
        scratch_types=[
            pltpu.VMEM((N,), jnp.int32),
            pltpu.VMEM((N + _L,), jnp.int32),
            pltpu.VMEM((_NBUF, EMB, _WIN), jnp.float32),
            pltpu.VMEM((_SLOTS, EMB), jnp.float32),
            pltpu.SemaphoreType.DMA,
            pltpu.SemaphoreType.DMA,
        ],
    )
    def _sc_gather(tt_hbm, idx_hbm, out_hbm, idx_all, owned, win, stag,
                   sem_w, sem_o):
        wid = lax.axis_index("s") * _NC + lax.axis_index("c")
        lo_w = wid * _WPT
        lanes = lax.iota(jnp.int32, _L)
        pltpu.sync_copy(idx_hbm, idx_all)

        def build(g, off):
            jv = lanes + g * _L
            rv = idx_all[pl.ds(g * _L, _L)]
            wv = rv // _WIN
            m = (wv >= lo_w) & (wv < lo_w + _WPT)
            plsc.store_compressed(owned.at[pl.ds(off, _L)], jv, mask=m)
            return off + jnp.sum(jnp.where(m, 1, 0))

        n_owned = lax.fori_loop(0, _NGRP, build, 0)
        n_grp = (n_owned + _L - 1) // _L

        def start_window(w):
            @pl.when(lo_w + w < _LAST_FULL)
            def _():
                pltpu.make_async_copy(
                    tt_hbm.at[:, pl.ds((lo_w + w) * _WIN, _WIN)],
                    win.at[w % _NBUF],
                    sem_w,
                ).start()

            @pl.when(lo_w + w == _LAST_FULL)
            def _():
                pltpu.make_async_copy(
                    tt_hbm.at[:, pl.ds((lo_w + w) * _WIN, 128)],
                    win.at[w % _NBUF, :, pl.ds(0, 128)],
                    sem_w,
                ).start()

        def wait_window(w):
            @pl.when(lo_w + w < _LAST_FULL)
            def _():
                pltpu.make_async_copy(
                    tt_hbm.at[:, pl.ds(0, _WIN)], win.at[w % _NBUF], sem_w
                ).wait()

            @pl.when(lo_w + w == _LAST_FULL)
            def _():
                pltpu.make_async_copy(
                    tt_hbm.at[:, pl.ds(0, 128)],
                    win.at[w % _NBUF, :, pl.ds(0, 128)],
                    sem_w,
                ).wait()

        start_window(0)
        start_window(1)

        def wloop(w, mc):
            wait_window(w)
            start_window(w + 2)

            def pgroup(g, mc):
                jv = owned[pl.ds(g * _L, _L)]
                vmask = (lanes + g * _L) < n_owned
                rv = plsc.load_gather(idx_all, [jnp.where(vmask, jv, 0)])
                m = (rv // _WIN == lo_w + w) & vmask

                def member(state):
                    m, mc = state
                    jsc = jnp.min(jnp.where(m, jv, jnp.int32(1 << 30)))
                    rsc = jnp.sum(jnp.where(jv == jsc, rv, 0))
                    lane = rsc % _WIN
                    slot = mc % _SLOTS
                    pv = jnp.full((_L,), w % _NBUF, jnp.int32)
                    lv = jnp.full((_L,), lane, jnp.int32)
                    for c4 in range(EMB // _L):
                        cv = lanes + c4 * _L
                        val = plsc.load_gather(win, [pv, cv, lv])
                        stag[slot, pl.ds(c4 * _L, _L)] = val
                    pltpu.make_async_copy(
                        stag.at[pl.ds(slot, 1)],
                        out_hbm.at[pl.ds(jsc, 1)],
                        sem_o,
                    ).start()

                    @pl.when(mc >= _LAG)
                    def _():
                        pltpu.make_async_copy(
                            stag.at[pl.ds(0, 1)],
                            out_hbm.at[pl.ds(0, 1)],
                            sem_o,
                        ).wait()

                    return m & (jv != jsc), mc + 1

                m, mc = lax.while_loop(
                    lambda s: jnp.any(s[0]), member, (m, mc))
                return mc

            return lax.fori_loop(0, n_grp, pgroup, mc)

        mc = lax.fori_loop(0, _WPT, wloop, 0)

        def drain(i, _):
            @pl.when(i < jnp.minimum(mc, _LAG))
            def _():
                pltpu.make_async_copy(
                    stag.at[pl.ds(0, 1)], out_hbm.at[pl.ds(0, 1)], sem_o
                ).wait()
            return 0

        lax.fori_loop(0, _LAG, drain, 0)

    return _sc_gather


_BLK = 2048


def _tc_body(base_ref, ecat_ref, charge_ref, l1k_ref, l1b_ref, l2k_ref,
             l2b_ref, proj_ref, out_ref):
    h = charge_ref[...] * l1k_ref[...] + l1b_ref[...]
    h = h * jax.nn.sigmoid(h)
    e_cont = jnp.dot(h, l2k_ref[...],
                     preferred_element_type=jnp.float32) + l2b_ref[...]
    p = proj_ref[...]
    out_ref[...] = (
        jnp.dot(base_ref[...], p[:BASE_DIM], preferred_element_type=jnp.float32)
        + jnp.dot(ecat_ref[...], p[BASE_DIM:BASE_DIM + EMB],
                  preferred_element_type=jnp.float32)
        + jnp.dot(e_cont, p[BASE_DIM + EMB:],
                  preferred_element_type=jnp.float32)
    )


def _tc_fused(base, e_cat, charge, l1k, l1b, l2k, l2b, proj):
    grid = N // _BLK
    return pl.pallas_call(
        _tc_body,
        grid=(grid,),
        in_specs=[
            pl.BlockSpec((_BLK, BASE_DIM), lambda i: (i, 0)),
            pl.BlockSpec((_BLK, EMB), lambda i: (i, 0)),
            pl.BlockSpec((_BLK, 1), lambda i: (i, 0)),
            pl.BlockSpec((1, EMB), lambda i: (0, 0)),
            pl.BlockSpec((1, EMB), lambda i: (0, 0)),
            pl.BlockSpec((EMB, EMB), lambda i: (0, 0)),
            pl.BlockSpec((1, EMB), lambda i: (0, 0)),
            pl.BlockSpec((BASE_DIM + 2 * EMB, OUT_DIM), lambda i: (0, 0)),
        ],
        out_specs=pl.BlockSpec((_BLK, OUT_DIM), lambda i: (i, 0)),
        out_shape=jax.ShapeDtypeStruct((N, OUT_DIM), jnp.float32),
    )(base, e_cat, charge, l1k, l1b, l2k, l2b, proj)


def kernel(base, element, charge, table, lin1_kernel, lin1_bias, lin2_kernel,
           lin2_bias, proj_kernel):
    el = element.astype(jnp.int32)
    e_cat = _make_sc_gather()(table.T, el)
    return _tc_fused(
        base, e_cat, charge,
        lin1_kernel.reshape(1, EMB), lin1_bias.reshape(1, EMB),
        lin2_kernel, lin2_bias.reshape(1, EMB), proj_kernel,
    )


# speedup vs baseline: 1.4002x; 1.4002x over previous
import functools

import jax
import jax.numpy as jnp
from jax import lax
from jax.experimental import pallas as pl
from jax.experimental.pallas import tpu as pltpu
from jax.experimental.pallas import tpu_sc as plsc

N = 16384
BASE_DIM = 128
VOCAB = 1000000
EMB = 64
OUT_DIM = 128

_NC = 2
_NS = 16
_NW = _NC * _NS
_L = 16
_WIN = 256
_NBUF = 4
_NWIN = (VOCAB + _WIN - 1) // _WIN
_WPT = (_NWIN + _NW - 1) // _NW
_LAST_FULL = (VOCAB + 127) // 128 * 128 // _WIN
_NGRP = N // _L
_SLOTS = 128
_LAG = 64


@functools.cache
def _make_sc_gather():
    @functools.partial(
        pl.kernel,
        mesh=plsc.VectorSubcoreMesh(core_axis_name="c", subcore_axis_name="s"),
        out_type=jax.ShapeDtypeStruct((N, EMB), jnp.float32),
        compiler_params=pltpu.CompilerParams(needs_layout_passes=False),
        scratch_types=[
            pltpu.VMEM((N,), jnp.int32),
            pltpu.VMEM((N + _L,), jnp.int32),
            pltpu.VMEM((_NBUF, EMB, _WIN), jnp.float32),
            pltpu.VMEM((_SLOTS, EMB), jnp.float32),
            pltpu.SemaphoreType.DMA,
            pltpu.SemaphoreType.DMA,
        ],
    )
    def _sc_gather(tt_hbm, idx_hbm, out_hbm, idx_all, owned, win, stag,
                   sem_w, sem_o):
        wid = lax.axis_index("s") * _NC + lax.axis_index("c")
        lo_w = wid * _WPT
        lanes = lax.iota(jnp.int32, _L)
        pltpu.sync_copy(idx_hbm, idx_all)

        def build(g, off):
            jv = lanes + g * _L
            rv = idx_all[pl.ds(g * _L, _L)]
            wv = rv // _WIN
            m = (wv >= lo_w) & (wv < lo_w + _WPT)
            plsc.store_compressed(owned.at[pl.ds(off, _L)], jv, mask=m)
            return off + jnp.sum(jnp.where(m, 1, 0))

        n_owned = lax.fori_loop(0, _NGRP, build, 0)
        n_grp = (n_owned + _L - 1) // _L

        def start_window(w):
            @pl.when(lo_w + w < _LAST_FULL)
            def _():
                pltpu.make_async_copy(
                    tt_hbm.at[:, pl.ds((lo_w + w) * _WIN, _WIN)],
                    win.at[w % _NBUF],
                    sem_w,
                ).start()

            @pl.when(lo_w + w == _LAST_FULL)
            def _():
                pltpu.make_async_copy(
                    tt_hbm.at[:, pl.ds((lo_w + w) * _WIN, 128)],
                    win.at[w % _NBUF, :, pl.ds(0, 128)],
                    sem_w,
                ).start()

        def wait_window(w):
            @pl.when(lo_w + w < _LAST_FULL)
            def _():
                pltpu.make_async_copy(
                    tt_hbm.at[:, pl.ds(0, _WIN)], win.at[w % _NBUF], sem_w
                ).wait()

            @pl.when(lo_w + w == _LAST_FULL)
            def _():
                pltpu.make_async_copy(
                    tt_hbm.at[:, pl.ds(0, 128)],
                    win.at[w % _NBUF, :, pl.ds(0, 128)],
                    sem_w,
                ).wait()

        start_window(0)
        start_window(1)

        def wloop(w, mc):
            wait_window(w)
            start_window(w + 2)

            def pgroup(g, mc):
                jv = owned[pl.ds(g * _L, _L)]
                vmask = (lanes + g * _L) < n_owned
                rv = plsc.load_gather(idx_all, [jnp.where(vmask, jv, 0)])
                m = (rv // _WIN == lo_w + w) & vmask

                def member(state):
                    m, mc = state
                    jsc = jnp.min(jnp.where(m, jv, jnp.int32(1 << 30)))
                    rsc = jnp.sum(jnp.where(jv == jsc, rv, 0))
                    lane = rsc % _WIN
                    slot = mc % _SLOTS
                    pv = jnp.full((_L,), w % _NBUF, jnp.int32)
                    lv = jnp.full((_L,), lane, jnp.int32)
                    for c4 in range(EMB // _L):
                        cv = lanes + c4 * _L
                        val = plsc.load_gather(win, [pv, cv, lv])
                        stag[slot, pl.ds(c4 * _L, _L)] = val
                    pltpu.make_async_copy(
                        stag.at[pl.ds(slot, 1)],
                        out_hbm.at[pl.ds(jsc, 1)],
                        sem_o,
                    ).start()

                    @pl.when(mc >= _LAG)
                    def _():
                        pltpu.make_async_copy(
                            stag.at[pl.ds(0, 1)],
                            out_hbm.at[pl.ds(0, 1)],
                            sem_o,
                        ).wait()

                    return m & (jv != jsc), mc + 1

                m, mc = lax.while_loop(
                    lambda s: jnp.any(s[0]), member, (m, mc))
                return mc

            return lax.fori_loop(0, n_grp, pgroup, mc)

        mc = lax.fori_loop(0, _WPT, wloop, 0)

        def drain(i, _):
            @pl.when(i < jnp.minimum(mc, _LAG))
            def _():
                pltpu.make_async_copy(
                    stag.at[pl.ds(0, 1)], out_hbm.at[pl.ds(0, 1)], sem_o
                ).wait()
            return 0

        lax.fori_loop(0, _LAG, drain, 0)

    return _sc_gather


_BLK = 2048


def _tc_body(base_ref, ecat_ref, charge_ref, l1k_ref, l1b_ref, l2k_ref,
             l2b_ref, proj_ref, out_ref):
    h = charge_ref[...] * l1k_ref[...] + l1b_ref[...]
    h = h * jax.nn.sigmoid(h)
    e_cont = jnp.dot(h, l2k_ref[...],
                     preferred_element_type=jnp.float32) + l2b_ref[...]
    p = proj_ref[...]
    out_ref[...] = (
        jnp.dot(base_ref[...], p[:BASE_DIM], preferred_element_type=jnp.float32)
        + jnp.dot(ecat_ref[...], p[BASE_DIM:BASE_DIM + EMB],
                  preferred_element_type=jnp.float32)
        + jnp.dot(e_cont, p[BASE_DIM + EMB:],
                  preferred_element_type=jnp.float32)
    )


def _tc_fused(base, e_cat, charge, l1k, l1b, l2k, l2b, proj):
    grid = N // _BLK
    return pl.pallas_call(
        _tc_body,
        grid=(grid,),
        in_specs=[
            pl.BlockSpec((_BLK, BASE_DIM), lambda i: (i, 0)),
            pl.BlockSpec((_BLK, EMB), lambda i: (i, 0)),
            pl.BlockSpec((_BLK, 1), lambda i: (i, 0)),
            pl.BlockSpec((1, EMB), lambda i: (0, 0)),
            pl.BlockSpec((1, EMB), lambda i: (0, 0)),
            pl.BlockSpec((EMB, EMB), lambda i: (0, 0)),
            pl.BlockSpec((1, EMB), lambda i: (0, 0)),
            pl.BlockSpec((BASE_DIM + 2 * EMB, OUT_DIM), lambda i: (0, 0)),
        ],
        out_specs=pl.BlockSpec((_BLK, OUT_DIM), lambda i: (i, 0)),
        out_shape=jax.ShapeDtypeStruct((N, OUT_DIM), jnp.float32),
    )(base, e_cat, charge, l1k, l1b, l2k, l2b, proj)


def kernel(base, element, charge, table, lin1_kernel, lin1_bias, lin2_kernel,
           lin2_bias, proj_kernel):
    el = element.astype(jnp.int32)
    e_cat = _make_sc_gather()(table.T, el)
    return _tc_fused(
        base, e_cat, charge,
        lin1_kernel.reshape(1, EMB), lin1_bias.reshape(1, EMB),
        lin2_kernel, lin2_bias.reshape(1, EMB), proj_kernel,
    )


# speedup vs baseline: 1.9853x; 1.4179x over previous
import functools

import jax
import jax.numpy as jnp
from jax import lax
from jax.experimental import pallas as pl
from jax.experimental.pallas import tpu as pltpu
from jax.experimental.pallas import tpu_sc as plsc

N = 16384
BASE_DIM = 128
VOCAB = 1000000
EMB = 64
OUT_DIM = 128

_NC = 2
_NS = 16
_NW = _NC * _NS
_L = 16
_WIN = 512
_NWIN = (VOCAB + _WIN - 1) // _WIN
_WPT = (_NWIN + _NW - 1) // _NW
_LAST_FULL = (VOCAB + 127) // 128 * 128 // _WIN
_REM_OFF = _LAST_FULL * _WIN
_NGRP = N // _L
_SLOTS = 128
_LAG = 64


@functools.cache
def _make_sc_gather():
    @functools.partial(
        pl.kernel,
        mesh=plsc.VectorSubcoreMesh(core_axis_name="c", subcore_axis_name="s"),
        out_type=jax.ShapeDtypeStruct((N, EMB), jnp.float32),
        compiler_params=pltpu.CompilerParams(needs_layout_passes=False),
        scratch_types=[
            pltpu.VMEM((N,), jnp.int32),
            pltpu.VMEM((N + _L,), jnp.int32),
            pltpu.VMEM((2, EMB, _WIN), jnp.float32),
            pltpu.VMEM((_SLOTS, EMB), jnp.float32),
            pltpu.SemaphoreType.DMA,
            pltpu.SemaphoreType.DMA,
        ],
    )
    def _sc_gather(tt_hbm, idx_hbm, out_hbm, idx_all, owned, win, stag,
                   sem_w, sem_o):
        wid = lax.axis_index("s") * _NC + lax.axis_index("c")
        lo_w = wid * _WPT
        lanes = lax.iota(jnp.int32, _L)
        pltpu.sync_copy(idx_hbm, idx_all)

        def build(g, off):
            jv = lanes + g * _L
            rv = idx_all[pl.ds(g * _L, _L)]
            wv = rv // _WIN
            m = (wv >= lo_w) & (wv < lo_w + _WPT)
            plsc.store_compressed(owned.at[pl.ds(off, _L)], jv, mask=m)
            return off + jnp.sum(jnp.where(m, 1, 0))

        n_owned = lax.fori_loop(0, _NGRP, build, 0)
        n_grp = (n_owned + _L - 1) // _L

        def start_window(w):
            @pl.when(lo_w + w < _LAST_FULL)
            def _():
                pltpu.make_async_copy(
                    tt_hbm.at[:, pl.ds((lo_w + w) * _WIN, _WIN)],
                    win.at[w % 2],
                    sem_w,
                ).start()

            @pl.when(lo_w + w == _LAST_FULL)
            def _():
                pltpu.make_async_copy(
                    tt_hbm.at[:, pl.ds((lo_w + w) * _WIN, 128)],
                    win.at[w % 2, :, pl.ds(0, 128)],
                    sem_w,
                ).start()

        def wait_window(w):
            @pl.when(lo_w + w < _LAST_FULL)
            def _():
                pltpu.make_async_copy(
                    tt_hbm.at[:, pl.ds(0, _WIN)], win.at[w % 2], sem_w
                ).wait()

            @pl.when(lo_w + w == _LAST_FULL)
            def _():
                pltpu.make_async_copy(
                    tt_hbm.at[:, pl.ds(0, 128)],
                    win.at[w % 2, :, pl.ds(0, 128)],
                    sem_w,
                ).wait()

        start_window(0)

        def wloop(w, mc):
            wait_window(w)
            start_window(w + 1)

            def pgroup(g, mc):
                jv = owned[pl.ds(g * _L, _L)]
                vmask = (lanes + g * _L) < n_owned
                rv = plsc.load_gather(idx_all, [jnp.where(vmask, jv, 0)])
                m = (rv // _WIN == lo_w + w) & vmask

                def member(state):
                    m, mc = state
                    jsc = jnp.min(jnp.where(m, jv, jnp.int32(1 << 30)))
                    rsc = jnp.sum(jnp.where(jv == jsc, rv, 0))
                    lane = rsc % _WIN
                    slot = mc % _SLOTS
                    pv = jnp.full((_L,), w % 2, jnp.int32)
                    lv = jnp.full((_L,), lane, jnp.int32)
                    for c4 in range(EMB // _L):
                        cv = lanes + c4 * _L
                        val = plsc.load_gather(win, [pv, cv, lv])
                        stag[slot, pl.ds(c4 * _L, _L)] = val
                    pltpu.make_async_copy(
                        stag.at[pl.ds(slot, 1)],
                        out_hbm.at[pl.ds(jsc, 1)],
                        sem_o,
                    ).start()

                    @pl.when(mc >= _LAG)
                    def _():
                        pltpu.make_async_copy(
                            stag.at[pl.ds(0, 1)],
                            out_hbm.at[pl.ds(0, 1)],
                            sem_o,
                        ).wait()

                    return m & (jv != jsc), mc + 1

                m, mc = lax.while_loop(
                    lambda s: jnp.any(s[0]), member, (m, mc))
                return mc

            return lax.fori_loop(0, n_grp, pgroup, mc)

        mc = lax.fori_loop(0, _WPT, wloop, 0)

        def drain(i, _):
            @pl.when(i < jnp.minimum(mc, _LAG))
            def _():
                pltpu.make_async_copy(
                    stag.at[pl.ds(0, 1)], out_hbm.at[pl.ds(0, 1)], sem_o
                ).wait()
            return 0

        lax.fori_loop(0, _LAG, drain, 0)

    return _sc_gather


_BLK = 2048


def _tc_body(base_ref, ecat_ref, charge_ref, l1k_ref, l1b_ref, l2k_ref,
             l2b_ref, proj_ref, out_ref):
    h = charge_ref[...] * l1k_ref[...] + l1b_ref[...]
    h = h * jax.nn.sigmoid(h)
    e_cont = jnp.dot(h, l2k_ref[...],
                     preferred_element_type=jnp.float32) + l2b_ref[...]
    p = proj_ref[...]
    out_ref[...] = (
        jnp.dot(base_ref[...], p[:BASE_DIM], preferred_element_type=jnp.float32)
        + jnp.dot(ecat_ref[...], p[BASE_DIM:BASE_DIM + EMB],
                  preferred_element_type=jnp.float32)
        + jnp.dot(e_cont, p[BASE_DIM + EMB:],
                  preferred_element_type=jnp.float32)
    )


def _tc_fused(base, e_cat, charge, l1k, l1b, l2k, l2b, proj):
    grid = N // _BLK
    return pl.pallas_call(
        _tc_body,
        grid=(grid,),
        in_specs=[
            pl.BlockSpec((_BLK, BASE_DIM), lambda i: (i, 0)),
            pl.BlockSpec((_BLK, EMB), lambda i: (i, 0)),
            pl.BlockSpec((_BLK, 1), lambda i: (i, 0)),
            pl.BlockSpec((1, EMB), lambda i: (0, 0)),
            pl.BlockSpec((1, EMB), lambda i: (0, 0)),
            pl.BlockSpec((EMB, EMB), lambda i: (0, 0)),
            pl.BlockSpec((1, EMB), lambda i: (0, 0)),
            pl.BlockSpec((BASE_DIM + 2 * EMB, OUT_DIM), lambda i: (0, 0)),
        ],
        out_specs=pl.BlockSpec((_BLK, OUT_DIM), lambda i: (i, 0)),
        out_shape=jax.ShapeDtypeStruct((N, OUT_DIM), jnp.float32),
    )(base, e_cat, charge, l1k, l1b, l2k, l2b, proj)


def kernel(base, element, charge, table, lin1_kernel, lin1_bias, lin2_kernel,
           lin2_bias, proj_kernel):
    el = element.astype(jnp.int32)
    e_cat = _make_sc_gather()(table.T, el)
    return _tc_fused(
        base, e_cat, charge,
        lin1_kernel.reshape(1, EMB), lin1_bias.reshape(1, EMB),
        lin2_kernel, lin2_bias.reshape(1, EMB), proj_kernel,
    )
